# baseline (speedup 1.0000x reference)
import jax
import jax.numpy as jnp
from jax.experimental import pallas as pl
from jax.experimental.pallas import tpu as pltpu

VMD_K = 8
HIDDEN = 128
INPUT_SIZE = 512
D = VMD_K * HIDDEN
STEPS = 2
K_PER_STEP = VMD_K // STEPS
D_PER_STEP = K_PER_STEP * HIDDEN


def _vmdk_kernel(in_ref, w_ref, b_ref, k0_ref, wp_ref, ow_ref, ob_ref,
                 out_ref, acc_ref):
    i = pl.program_id(0)
    m = jax.lax.dot_general(
        in_ref[:], w_ref[:],
        dimension_numbers=(((1,), (1,)), ((), ())),
        preferred_element_type=jnp.float32,
    )
    kk = jax.lax.broadcasted_iota(jnp.int32, (VMD_K, D_PER_STEP), 0)
    jc = jax.lax.broadcasted_iota(jnp.int32, (VMD_K, D_PER_STEP), 1) // HIDDEN
    mask = kk == jc + i * K_PER_STEP
    b_tiled = jnp.concatenate([b_ref[:]] * K_PER_STEP, axis=1)
    hm = jnp.sum(jnp.where(mask, m + b_tiled, 0.0), axis=0, keepdims=True)
    h = jnp.maximum(hm, 0.0)
    sel = (h - k0_ref[0:1, :]) * wp_ref[:]
    part = sel * ow_ref[:]

    @pl.when(i == 0)
    def _():
        acc_ref[:] = part

    @pl.when(i > 0)
    def _():
        val = jnp.sum(acc_ref[:] + part, axis=(0, 1), keepdims=True) + ob_ref[:]
        out_ref[:] = 1.0 / (1.0 + jnp.exp(-val))


def kernel(input, vmd_w, vmd_b, K_param, W_param, out_w, out_b):
    vmd_w2 = vmd_w.reshape(D, INPUT_SIZE)
    out = pl.pallas_call(
        _vmdk_kernel,
        grid=(STEPS,),
        in_specs=[
            pl.BlockSpec((VMD_K, INPUT_SIZE), lambda i: (0, 0)),
            pl.BlockSpec((D_PER_STEP, INPUT_SIZE), lambda i: (i, 0)),
            pl.BlockSpec((VMD_K, HIDDEN), lambda i: (0, 0)),
            pl.BlockSpec((8, D_PER_STEP), lambda i: (0, i)),
            pl.BlockSpec((1, D_PER_STEP), lambda i: (0, i)),
            pl.BlockSpec((1, D_PER_STEP), lambda i: (0, i)),
            pl.BlockSpec((1, 1), lambda i: (0, 0)),
        ],
        out_specs=pl.BlockSpec((1, 1), lambda i: (0, 0)),
        out_shape=jax.ShapeDtypeStruct((1, 1), jnp.float32),
        scratch_shapes=[pltpu.VMEM((1, D_PER_STEP), jnp.float32)],
    )(
        input, vmd_w2, vmd_b, K_param,
        W_param.reshape(1, D),
        out_w,
        out_b.reshape(1, 1),
    )
    return out.reshape(1)


# speedup vs baseline: 1.0106x; 1.0106x over previous
import jax
import jax.numpy as jnp
from jax.experimental import pallas as pl
from jax.experimental.pallas import tpu as pltpu

VMD_K = 8
HIDDEN = 128
INPUT_SIZE = 512
D = VMD_K * HIDDEN
STEPS = 2
K_PER_STEP = VMD_K // STEPS
D_PER_STEP = K_PER_STEP * HIDDEN


def _vmdk_kernel(in_ref, w_ref, b_ref, k0_ref, wp_ref, ow_ref, ob_ref,
                 out_ref, acc_ref):
    i = pl.program_id(0)
    acc = jnp.zeros((1, HIDDEN), jnp.float32)
    for k in range(K_PER_STEP):
        lo = k * HIDDEN
        hm = jax.lax.dot_general(
            in_ref[pl.ds(i * K_PER_STEP + k, 1), :], w_ref[k],
            dimension_numbers=(((1,), (1,)), ((), ())),
            preferred_element_type=jnp.float32,
        )
        h = jnp.maximum(hm + b_ref[pl.ds(i * K_PER_STEP + k, 1), :], 0.0)
        sel = (h - k0_ref[0:1, lo:lo + HIDDEN]) * wp_ref[:, lo:lo + HIDDEN]
        acc = acc + sel * ow_ref[:, lo:lo + HIDDEN]

    @pl.when(i == 0)
    def _():
        acc_ref[:] = acc

    @pl.when(i > 0)
    def _():
        val = jnp.sum(acc_ref[:] + acc, axis=(0, 1), keepdims=True) + ob_ref[:]
        out_ref[:] = 1.0 / (1.0 + jnp.exp(-val))


def kernel(input, vmd_w, vmd_b, K_param, W_param, out_w, out_b):
    out = pl.pallas_call(
        _vmdk_kernel,
        grid=(STEPS,),
        in_specs=[
            pl.BlockSpec((VMD_K, INPUT_SIZE), lambda i: (0, 0)),
            pl.BlockSpec((K_PER_STEP, HIDDEN, INPUT_SIZE), lambda i: (i, 0, 0)),
            pl.BlockSpec((VMD_K, HIDDEN), lambda i: (0, 0)),
            pl.BlockSpec((8, D_PER_STEP), lambda i: (0, i)),
            pl.BlockSpec((1, D_PER_STEP), lambda i: (0, i)),
            pl.BlockSpec((1, D_PER_STEP), lambda i: (0, i)),
            pl.BlockSpec((1, 1), lambda i: (0, 0)),
        ],
        out_specs=pl.BlockSpec((1, 1), lambda i: (0, 0)),
        out_shape=jax.ShapeDtypeStruct((1, 1), jnp.float32),
        scratch_shapes=[pltpu.VMEM((1, HIDDEN), jnp.float32)],
    )(
        input, vmd_w, vmd_b, K_param,
        W_param.reshape(1, D),
        out_w,
        out_b.reshape(1, 1),
    )
    return out.reshape(1)
